# baseline (speedup 1.0000x reference)
import functools

import jax
import jax.numpy as jnp
from jax import lax
from jax.experimental import pallas as pl
from jax.experimental.pallas import tpu as pltpu
from jax.experimental.pallas import tpu_sc as plsc

D_MODEL = 64
NUM_WORKERS = 32
ROWS_PER_GATHER = 128
GATHERS_PER_GROUP = 5
GROUP = ROWS_PER_GATHER * GATHERS_PER_GROUP


@functools.lru_cache(maxsize=None)
def _make_kernel(B):
    rows_per_w = B // NUM_WORKERS
    groups_per_w = rows_per_w // GROUP
    assert rows_per_w % GROUP == 0 and groups_per_w % 2 == 0
    mesh = plsc.VectorSubcoreMesh(core_axis_name="c", subcore_axis_name="s")

    @functools.partial(
        pl.kernel,
        out_type=jax.ShapeDtypeStruct((B, D_MODEL), jnp.float32),
        mesh=mesh,
        scratch_types=[
            pltpu.VMEM((GATHERS_PER_GROUP, ROWS_PER_GATHER), jnp.int32),
            pltpu.VMEM((GATHERS_PER_GROUP, ROWS_PER_GATHER), jnp.int32),
            pltpu.VMEM((GROUP, D_MODEL), jnp.float32),
            pltpu.VMEM((GROUP, D_MODEL), jnp.float32),
            pltpu.SemaphoreType.DMA,
            pltpu.SemaphoreType.DMA,
        ],
        compiler_params=pltpu.CompilerParams(use_tc_tiling_on_sc=False),
    )
    def gather_kernel(idx_hbm, table_hbm, out_hbm, idx_a, idx_b,
                      rows_a, rows_b, sem_a, sem_b):
        wid = lax.axis_index("s") * 2 + lax.axis_index("c")
        w_base = wid * rows_per_w

        def gbase_of(g):
            return pl.multiple_of(w_base + g * GROUP, GROUP)

        def fire(g, idx_v, rows_v, sem):
            irow = pl.multiple_of(gbase_of(g) // ROWS_PER_GATHER,
                                  GATHERS_PER_GROUP)
            pltpu.sync_copy(idx_hbm.at[pl.ds(irow, GATHERS_PER_GROUP), :],
                            idx_v)
            for j in range(GATHERS_PER_GROUP):
                pltpu.async_copy(
                    table_hbm.at[idx_v.at[j]],
                    rows_v.at[pl.ds(j * ROWS_PER_GATHER, ROWS_PER_GATHER)],
                    sem)

        def drain_store(g, rows_v, sem):
            pltpu.make_async_copy(out_hbm.at[pl.ds(0, GROUP)], rows_v,
                                  sem).wait()
            pltpu.sync_copy(rows_v, out_hbm.at[pl.ds(gbase_of(g), GROUP)])

        fire(0, idx_a, rows_a, sem_a)

        def body(i, carry):
            g0 = 2 * i
            fire(g0 + 1, idx_b, rows_b, sem_b)
            drain_store(g0, rows_a, sem_a)

            @pl.when(i < groups_per_w // 2 - 1)
            def _():
                fire(g0 + 2, idx_a, rows_a, sem_a)

            drain_store(g0 + 1, rows_b, sem_b)
            return carry

        lax.fori_loop(0, groups_per_w // 2, body, 0)

    return gather_kernel


def kernel(tf_indices, table):
    batch, hist = tf_indices.shape
    B = batch * hist
    idx2 = tf_indices.reshape(B // ROWS_PER_GATHER, ROWS_PER_GATHER)
    idx2 = idx2.astype(jnp.int32)
    out = _make_kernel(B)(idx2, table)
    return out.reshape(batch, hist, D_MODEL)


# speedup vs baseline: 1.7669x; 1.7669x over previous
import functools

import jax
import jax.numpy as jnp
from jax import lax
from jax.experimental import pallas as pl
from jax.experimental.pallas import tpu as pltpu
from jax.experimental.pallas import tpu_sc as plsc

N_ROWS = 1000
D_MODEL = 64
NUM_WORKERS = 32
CHUNK_B = 256
H_OCT = 8


@functools.lru_cache(maxsize=None)
def _make_kernel(batch, hist):
    chunks = batch // CHUNK_B
    chunks_per_w = chunks // NUM_WORKERS
    octets = hist // H_OCT
    assert batch % (CHUNK_B * NUM_WORKERS) == 0 and hist % H_OCT == 0
    mesh = plsc.VectorSubcoreMesh(core_axis_name="c", subcore_axis_name="s")

    @functools.partial(
        pl.kernel,
        out_type=jax.ShapeDtypeStruct((hist, D_MODEL, batch), jnp.float32),
        mesh=mesh,
        scratch_types=[
            pltpu.VMEM((N_ROWS * D_MODEL,), jnp.float32),
            pltpu.VMEM((H_OCT, CHUNK_B), jnp.int32),
            pltpu.VMEM((D_MODEL, CHUNK_B), jnp.float32),
            pltpu.VMEM((D_MODEL, CHUNK_B), jnp.float32),
            pltpu.SemaphoreType.DMA,
            pltpu.SemaphoreType.DMA,
        ],
        compiler_params=pltpu.CompilerParams(use_tc_tiling_on_sc=True,
                                             needs_layout_passes=False),
    )
    def gather_kernel(idx_hbm, table_hbm, out_hbm, table_v, idx_v,
                      stage_a, stage_b, sem_a, sem_b):
        wid = lax.axis_index("s") * 2 + lax.axis_index("c")
        pltpu.sync_copy(table_hbm, table_v)

        stages = (stage_a, stage_b)
        sems = (sem_a, sem_b)

        for c in range(chunks_per_w):
            b0 = pl.multiple_of((wid * chunks_per_w + c) * CHUNK_B, CHUNK_B)

            def octet_body(o, carry, c=c):
                h0 = pl.multiple_of(o * H_OCT, H_OCT)
                pltpu.sync_copy(
                    idx_hbm.at[pl.ds(h0, H_OCT), pl.ds(b0, CHUNK_B)], idx_v)
                for hh in range(H_OCT):
                    p = hh % 2
                    stage, sem = stages[p], sems[p]

                    def drain(stage=stage, sem=sem):
                        pltpu.make_async_copy(
                            out_hbm.at[0, :, pl.ds(0, CHUNK_B)], stage,
                            sem).wait()

                    if hh >= 2 or c > 0:
                        drain()
                    else:
                        pl.when(o > 0)(drain)

                    def bv_body(bv, carry2, stage=stage):
                        iv = idx_v[hh, pl.ds(bv * 16, 16)]
                        for d in range(D_MODEL):
                            vals = plsc.load_gather(table_v, [iv + d * N_ROWS])
                            stage[d, pl.ds(bv * 16, 16)] = vals
                        return carry2

                    lax.fori_loop(0, CHUNK_B // 16, bv_body, 0)
                    pltpu.async_copy(
                        stage, out_hbm.at[h0 + hh, :, pl.ds(b0, CHUNK_B)],
                        sem)
                return carry

            lax.fori_loop(0, octets, octet_body, 0)

        for p in range(2):
            pltpu.make_async_copy(
                out_hbm.at[0, :, pl.ds(0, CHUNK_B)], stages[p],
                sems[p]).wait()

    return gather_kernel


def kernel(tf_indices, table):
    batch, hist = tf_indices.shape
    idx_t = tf_indices.T.astype(jnp.int32)
    table_t = table.T.reshape(D_MODEL * N_ROWS)
    out_t = _make_kernel(batch, hist)(idx_t, table_t)
    return out_t.transpose(2, 0, 1)


# speedup vs baseline: 5.7887x; 3.2762x over previous
import functools

import jax
import jax.numpy as jnp
from jax import lax
from jax.experimental import pallas as pl
from jax.experimental.pallas import tpu as pltpu
from jax.experimental.pallas import tpu_sc as plsc

N_ROWS = 1000
D_MODEL = 64
NUM_WORKERS = 32
CHUNK_B = 256
H_OCT = 8


@functools.lru_cache(maxsize=None)
def _make_kernel(batch, hist):
    chunks = batch // CHUNK_B
    chunks_per_w = chunks // NUM_WORKERS
    octets = hist // H_OCT
    assert batch % (CHUNK_B * NUM_WORKERS) == 0 and hist % H_OCT == 0
    mesh = plsc.VectorSubcoreMesh(core_axis_name="c", subcore_axis_name="s")

    @functools.partial(
        pl.kernel,
        out_type=jax.ShapeDtypeStruct((hist, D_MODEL, batch), jnp.float32),
        mesh=mesh,
        scratch_types=[
            pltpu.VMEM((N_ROWS * D_MODEL,), jnp.float32),
            pltpu.VMEM((H_OCT, CHUNK_B), jnp.int32),
            pltpu.VMEM((D_MODEL, CHUNK_B), jnp.float32),
            pltpu.VMEM((D_MODEL, CHUNK_B), jnp.float32),
            pltpu.SemaphoreType.DMA,
            pltpu.SemaphoreType.DMA,
        ],
        compiler_params=pltpu.CompilerParams(use_tc_tiling_on_sc=True,
                                             needs_layout_passes=False),
    )
    def gather_kernel(idx_hbm, table_hbm, out_hbm, table_v, idx_v,
                      stage_a, stage_b, sem_a, sem_b):
        wid = lax.axis_index("s") * 2 + lax.axis_index("c")
        pltpu.sync_copy(table_hbm, table_v)

        stages = (stage_a, stage_b)
        sems = (sem_a, sem_b)

        for c in range(chunks_per_w):
            b0 = pl.multiple_of((wid * chunks_per_w + c) * CHUNK_B, CHUNK_B)

            def octet_body(o, carry, c=c):
                h0 = pl.multiple_of(o * H_OCT, H_OCT)
                pltpu.sync_copy(
                    idx_hbm.at[pl.ds(h0, H_OCT), pl.ds(b0, CHUNK_B)], idx_v)
                for hh in range(H_OCT):
                    p = hh % 2
                    stage, sem = stages[p], sems[p]

                    def drain(stage=stage, sem=sem):
                        pltpu.make_async_copy(
                            out_hbm.at[0, :, pl.ds(0, CHUNK_B)], stage,
                            sem).wait()

                    if hh >= 2 or c > 0:
                        drain()
                    else:
                        pl.when(o > 0)(drain)

                    def bv_body(bv, carry2, stage=stage):
                        G = 8
                        iv = idx_v[hh, pl.ds(bv * 16, 16)]
                        prev = None
                        for dg in range(0, D_MODEL, G):
                            cur = [
                                (d, plsc.load_gather(table_v,
                                                     [iv + d * N_ROWS]))
                                for d in range(dg, dg + G)
                            ]
                            if prev is not None:
                                for d, vals in prev:
                                    stage[d, pl.ds(bv * 16, 16)] = vals
                            prev = cur
                        for d, vals in prev:
                            stage[d, pl.ds(bv * 16, 16)] = vals
                        return carry2

                    lax.fori_loop(0, CHUNK_B // 16, bv_body, 0)
                    pltpu.async_copy(
                        stage, out_hbm.at[h0 + hh, :, pl.ds(b0, CHUNK_B)],
                        sem)
                return carry

            lax.fori_loop(0, octets, octet_body, 0)

        for p in range(2):
            pltpu.make_async_copy(
                out_hbm.at[0, :, pl.ds(0, CHUNK_B)], stages[p],
                sems[p]).wait()

    return gather_kernel


def kernel(tf_indices, table):
    batch, hist = tf_indices.shape
    idx_t = tf_indices.T.astype(jnp.int32)
    table_t = table.T.reshape(D_MODEL * N_ROWS)
    out_t = _make_kernel(batch, hist)(idx_t, table_t)
    return out_t.transpose(2, 0, 1)


# speedup vs baseline: 5.8039x; 1.0026x over previous
import functools

import jax
import jax.numpy as jnp
from jax import lax
from jax.experimental import pallas as pl
from jax.experimental.pallas import tpu as pltpu
from jax.experimental.pallas import tpu_sc as plsc

N_ROWS = 1000
D_MODEL = 64
NUM_WORKERS = 32
CHUNK_B = 256
H_OCT = 8


@functools.lru_cache(maxsize=None)
def _make_kernel(batch, hist):
    chunks = batch // CHUNK_B
    chunks_per_w = chunks // NUM_WORKERS
    octets = hist // H_OCT
    assert batch % (CHUNK_B * NUM_WORKERS) == 0 and hist % H_OCT == 0
    mesh = plsc.VectorSubcoreMesh(core_axis_name="c", subcore_axis_name="s")

    @functools.partial(
        pl.kernel,
        out_type=jax.ShapeDtypeStruct((hist, D_MODEL, batch), jnp.float32),
        mesh=mesh,
        scratch_types=[
            pltpu.VMEM((N_ROWS * D_MODEL,), jnp.float32),
            pltpu.VMEM((H_OCT, CHUNK_B), jnp.int32),
            pltpu.VMEM((D_MODEL, CHUNK_B), jnp.float32),
            pltpu.VMEM((D_MODEL, CHUNK_B), jnp.float32),
            pltpu.VMEM((D_MODEL, CHUNK_B), jnp.float32),
            pltpu.SemaphoreType.DMA,
            pltpu.SemaphoreType.DMA,
            pltpu.SemaphoreType.DMA,
        ],
        compiler_params=pltpu.CompilerParams(use_tc_tiling_on_sc=True,
                                             needs_layout_passes=False),
    )
    def gather_kernel(idx_hbm, table_hbm, out_hbm, table_v, idx_v,
                      stage_a, stage_b, stage_c, sem_a, sem_b, sem_c):
        wid = lax.axis_index("s") * 2 + lax.axis_index("c")
        pltpu.sync_copy(table_hbm, table_v)

        stages = (stage_a, stage_b, stage_c)
        sems = (sem_a, sem_b, sem_c)
        depth = len(stages)

        for c in range(chunks_per_w):
            b0 = pl.multiple_of((wid * chunks_per_w + c) * CHUNK_B, CHUNK_B)

            def octet_body(o, carry, c=c):
                h0 = pl.multiple_of(o * H_OCT, H_OCT)
                pltpu.sync_copy(
                    idx_hbm.at[pl.ds(h0, H_OCT), pl.ds(b0, CHUNK_B)], idx_v)
                for hh in range(H_OCT):
                    p = hh % depth
                    stage, sem = stages[p], sems[p]

                    def drain(stage=stage, sem=sem):
                        pltpu.make_async_copy(
                            out_hbm.at[0, :, pl.ds(0, CHUNK_B)], stage,
                            sem).wait()

                    if hh >= depth or c > 0:
                        drain()
                    else:
                        pl.when(o > 0)(drain)

                    def bv_body(bv, carry2, stage=stage):
                        G = 8
                        iv = idx_v[hh, pl.ds(bv * 16, 16)]
                        prev = None
                        for dg in range(0, D_MODEL, G):
                            cur = [
                                (d, plsc.load_gather(table_v,
                                                     [iv + d * N_ROWS]))
                                for d in range(dg, dg + G)
                            ]
                            if prev is not None:
                                for d, vals in prev:
                                    stage[d, pl.ds(bv * 16, 16)] = vals
                            prev = cur
                        for d, vals in prev:
                            stage[d, pl.ds(bv * 16, 16)] = vals
                        return carry2

                    lax.fori_loop(0, CHUNK_B // 16, bv_body, 0)
                    pltpu.async_copy(
                        stage, out_hbm.at[h0 + hh, :, pl.ds(b0, CHUNK_B)],
                        sem)
                return carry

            lax.fori_loop(0, octets, octet_body, 0)

        for p in range(depth):
            pltpu.make_async_copy(
                out_hbm.at[0, :, pl.ds(0, CHUNK_B)], stages[p],
                sems[p]).wait()

    return gather_kernel


def kernel(tf_indices, table):
    batch, hist = tf_indices.shape
    idx_t = tf_indices.T.astype(jnp.int32)
    table_t = table.T.reshape(D_MODEL * N_ROWS)
    out_t = _make_kernel(batch, hist)(idx_t, table_t)
    return out_t.transpose(2, 0, 1)
